# baseline (speedup 1.0000x reference)
import functools

import jax
import jax.numpy as jnp
from jax import lax
from jax.experimental import pallas as pl
from jax.experimental.pallas import tpu as pltpu
from jax.experimental.pallas import tpu_sc as plsc

F = 16
B = 16384
NU = 1000000
NI = 100000
NC = 2
NS = 16
NW = NC * NS
BPW = B // NW
CHB = 128
NCH = BPW // CHB
CCK = CHB // F

TCC = 262144
NUB = (NU + TCC - 1) // TCC
NUP = NUB * TCC


def _repack_body(a_ref, *out_refs):
    for r in range(8):
        out_refs[r][...] = a_ref[r, :]


def _repack1(a_t):
    blk = lambda: pl.BlockSpec((TCC,), lambda kk, j: (kk * NUB + j,))
    return pl.pallas_call(
        _repack_body,
        grid=(2, NUB),
        in_specs=[pl.BlockSpec((8, TCC), lambda kk, j: (kk, j))],
        out_specs=[blk() for _ in range(8)],
        out_shape=[jax.ShapeDtypeStruct((2 * NUP,), jnp.float32)
                   for _ in range(8)],
    )(a_t)


def _sc_items_body(items_h, ig_h, im_h,
                   igr_o, im_o,
                   iidx, kidxi, ig_p, im_p, ig_s, im_s, sem):
    wid = lax.axis_index("s") * NC + lax.axis_index("c")
    base = wid * BPW
    pltpu.sync_copy(items_h.at[pl.ds(base, BPW)], iidx)
    rowoff = lax.iota(jnp.int32, F) * CHB

    for c in range(NCH):
        r0 = c * CHB

        def bump(g, carry, r0=r0):
            it = iidx[pl.ds(r0 + g * F, F)]
            for k in range(F):
                kidxi[pl.ds(k * CHB + g * F, F)] = it + (k * NI)
            return carry

        lax.fori_loop(0, CCK, bump, 0)
        c0 = pltpu.async_copy(ig_h.at[kidxi], ig_p, sem)
        c1 = pltpu.async_copy(im_h.at[kidxi], im_p, sem)
        c0.wait()
        c1.wait()

        def asm(i, carry):
            idxv = rowoff + i
            ig_s[i] = plsc.load_gather(ig_p, [idxv])
            im_s[i] = plsc.load_gather(im_p, [idxv])
            return carry

        lax.fori_loop(0, CHB, asm, 0)
        pltpu.sync_copy(ig_s, igr_o.at[pl.ds(base + r0, CHB)])
        pltpu.sync_copy(im_s, im_o.at[pl.ds(base + r0, CHB)])


_sc_items = functools.partial(
    pl.kernel,
    mesh=plsc.VectorSubcoreMesh(core_axis_name="c", subcore_axis_name="s"),
    compiler_params=pltpu.CompilerParams(
        needs_layout_passes=False, use_tc_tiling_on_sc=False),
    out_type=[
        jax.ShapeDtypeStruct((B, F), jnp.float32),
        jax.ShapeDtypeStruct((B, F), jnp.float32),
    ],
    scratch_types=[
        pltpu.VMEM((BPW,), jnp.int32),
        pltpu.VMEM((F * CHB,), jnp.int32),
        pltpu.VMEM((F * CHB,), jnp.float32),
        pltpu.VMEM((F * CHB,), jnp.float32),
        pltpu.VMEM((CHB, F), jnp.float32),
        pltpu.VMEM((CHB, F), jnp.float32),
        pltpu.SemaphoreType.DMA,
    ],
)(_sc_items_body)


def _user_gather_loop(users_h, planes, uidx, kidxu1, bufs, sems, emit):
    def bump(g, carry):
        s = pl.ds(g * F, F)
        kidxu1[s] = uidx[s] + NUP
        return carry

    lax.fori_loop(0, BPW // F, bump, 0)

    def fire(c):
        b = c % 2
        r0 = c * CHB
        u0 = uidx.at[pl.ds(r0, CHB)]
        u1 = kidxu1.at[pl.ds(r0, CHB)]
        cs = []
        for r in range(8):
            cs.append(pltpu.async_copy(
                planes[r].at[u0], bufs[b].at[pl.ds(r * CHB, CHB)], sems[b]))
            cs.append(pltpu.async_copy(
                planes[r].at[u1],
                bufs[b].at[pl.ds((8 + r) * CHB, CHB)], sems[b]))
        return cs

    pending = fire(0)
    for c in range(NCH):
        for cp in pending:
            cp.wait()
        if c + 1 < NCH:
            pending = fire(c + 1)
        emit(c, c % 2)


def _sc_ug_body(users_h, igr_h, *rest):
    planes = rest[0:8]
    gmf_o = rest[8]
    (uidx, kidxu1, p0, p1, ig_r, gm_s, sem0, sem1) = rest[9:]
    bufs, sems = (p0, p1), (sem0, sem1)

    wid = lax.axis_index("s") * NC + lax.axis_index("c")
    base = wid * BPW
    pltpu.sync_copy(users_h.at[pl.ds(base, BPW)], uidx)
    rowoff = lax.iota(jnp.int32, F) * CHB

    def emit(c, b):
        r0 = c * CHB
        pltpu.sync_copy(igr_h.at[pl.ds(base + r0, CHB)], ig_r)

        def asm(i, carry, b=b):
            ug = plsc.load_gather(bufs[b], [rowoff + i])
            gm_s[i] = ug * ig_r[i]
            return carry

        lax.fori_loop(0, CHB, asm, 0)
        pltpu.sync_copy(gm_s, gmf_o.at[pl.ds(base + r0, CHB)])

    _user_gather_loop(users_h, planes, uidx, kidxu1, bufs, sems, emit)


def _sc_um_body(users_h, *rest):
    planes = rest[0:8]
    um_o = rest[8]
    (uidx, kidxu1, p0, p1, um_s, sem0, sem1) = rest[9:]
    bufs, sems = (p0, p1), (sem0, sem1)

    wid = lax.axis_index("s") * NC + lax.axis_index("c")
    base = wid * BPW
    pltpu.sync_copy(users_h.at[pl.ds(base, BPW)], uidx)
    rowoff = lax.iota(jnp.int32, F) * CHB

    def emit(c, b):
        r0 = c * CHB

        def asm(i, carry, b=b):
            um_s[i] = plsc.load_gather(bufs[b], [rowoff + i])
            return carry

        lax.fori_loop(0, CHB, asm, 0)
        pltpu.sync_copy(um_s, um_o.at[pl.ds(base + r0, CHB)])

    _user_gather_loop(users_h, planes, uidx, kidxu1, bufs, sems, emit)


_ug_scratch = [
    pltpu.VMEM((BPW,), jnp.int32),
    pltpu.VMEM((BPW,), jnp.int32),
    pltpu.VMEM((F * CHB,), jnp.float32),
    pltpu.VMEM((F * CHB,), jnp.float32),
    pltpu.VMEM((CHB, F), jnp.float32),
    pltpu.VMEM((CHB, F), jnp.float32),
    pltpu.SemaphoreType.DMA,
    pltpu.SemaphoreType.DMA,
]

_sc_ug = functools.partial(
    pl.kernel,
    mesh=plsc.VectorSubcoreMesh(core_axis_name="c", subcore_axis_name="s"),
    compiler_params=pltpu.CompilerParams(
        needs_layout_passes=False, use_tc_tiling_on_sc=False),
    out_type=[jax.ShapeDtypeStruct((B, F), jnp.float32)],
    scratch_types=list(_ug_scratch),
)(_sc_ug_body)

_sc_um = functools.partial(
    pl.kernel,
    mesh=plsc.VectorSubcoreMesh(core_axis_name="c", subcore_axis_name="s"),
    compiler_params=pltpu.CompilerParams(
        needs_layout_passes=False, use_tc_tiling_on_sc=False),
    out_type=[jax.ShapeDtypeStruct((B, F), jnp.float32)],
    scratch_types=[
        pltpu.VMEM((BPW,), jnp.int32),
        pltpu.VMEM((BPW,), jnp.int32),
        pltpu.VMEM((F * CHB,), jnp.float32),
        pltpu.VMEM((F * CHB,), jnp.float32),
        pltpu.VMEM((CHB, F), jnp.float32),
        pltpu.SemaphoreType.DMA,
        pltpu.SemaphoreType.DMA,
    ],
)(_sc_um_body)


BM = 2048


def _tc_body(gmf_ref, um_ref, im_ref, w1_ref, b1_ref, w2_ref, b2_ref,
             wog_ref, woh_ref, bo_ref, out_ref):
    mlp_in = jnp.concatenate([um_ref[...], im_ref[...]], axis=1)
    h = jnp.dot(mlp_in, w1_ref[...], preferred_element_type=jnp.float32)
    h = jnp.maximum(h + b1_ref[...], 0.0)
    h = jnp.dot(h, w2_ref[...], preferred_element_type=jnp.float32)
    h = jnp.maximum(h + b2_ref[...], 0.0)
    s = jnp.dot(gmf_ref[...], wog_ref[...], preferred_element_type=jnp.float32)
    s = s + jnp.dot(h, woh_ref[...], preferred_element_type=jnp.float32)
    out_ref[...] = s + bo_ref[...]


def _tc_mlp(gmf, um, im, W1, b1, W2, b2, Wo, bo):
    grid = (B // BM,)
    full = lambda shape: pl.BlockSpec(shape, lambda i: (0, 0))
    return pl.pallas_call(
        _tc_body,
        grid=grid,
        in_specs=[
            pl.BlockSpec((BM, F), lambda i: (i, 0)),
            pl.BlockSpec((BM, F), lambda i: (i, 0)),
            pl.BlockSpec((BM, F), lambda i: (i, 0)),
            full((2 * F, 2 * F)),
            full((1, 2 * F)),
            full((2 * F, F)),
            full((1, F)),
            full((F, 1)),
            full((F, 1)),
            full((1, 1)),
        ],
        out_specs=pl.BlockSpec((BM, 1), lambda i: (i, 0)),
        out_shape=jax.ShapeDtypeStruct((B, 1), jnp.float32),
    )(gmf, um, im, W1, b1.reshape(1, -1), W2, b2.reshape(1, -1),
      Wo[:F], Wo[F:], bo.reshape(1, 1))


def kernel(users, items, user_gmf, item_gmf, user_mlp, item_mlp,
           W1, b1, W2, b2, Wo, bo):
    users = users.astype(jnp.int32)
    items = items.astype(jnp.int32)
    ig_f = item_gmf.T.reshape(-1)
    im_f = item_mlp.T.reshape(-1)
    igr, im = _sc_items(items, ig_f, im_f)
    ug_planes = _repack1(user_gmf.T)
    gmf = _sc_ug(users, igr, *ug_planes)
    um_planes = _repack1(user_mlp.T)
    um = _sc_um(users, *um_planes)
    if isinstance(gmf, (list, tuple)):
        gmf = gmf[0]
    if isinstance(um, (list, tuple)):
        um = um[0]
    scores = _tc_mlp(gmf, um, im, W1, b1, W2, b2, Wo, bo)
    return scores[:, 0]


# speedup vs baseline: 1.0009x; 1.0009x over previous
import functools

import jax
import jax.numpy as jnp
from jax import lax
from jax.experimental import pallas as pl
from jax.experimental.pallas import tpu as pltpu
from jax.experimental.pallas import tpu_sc as plsc

F = 16
B = 16384
NU = 1000000
NI = 100000
NC = 2
NS = 16
NW = NC * NS
BPW = B // NW
CHB = 128
NCH = BPW // CHB
CCK = CHB // F

TCC = 131072
NUB = (NU + TCC - 1) // TCC
NUP = NUB * TCC


def _repack_body(a_ref, *out_refs):
    for r in range(8):
        out_refs[r][...] = a_ref[r, :]


def _repack1(a_t):
    blk = lambda: pl.BlockSpec((TCC,), lambda kk, j: (kk * NUB + j,))
    return pl.pallas_call(
        _repack_body,
        grid=(2, NUB),
        in_specs=[pl.BlockSpec((8, TCC), lambda kk, j: (kk, j))],
        out_specs=[blk() for _ in range(8)],
        out_shape=[jax.ShapeDtypeStruct((2 * NUP,), jnp.float32)
                   for _ in range(8)],
    )(a_t)


def _sc_items_body(items_h, ig_h, im_h,
                   igr_o, im_o,
                   iidx, kidxi, ig_p, im_p, ig_s, im_s, sem):
    wid = lax.axis_index("s") * NC + lax.axis_index("c")
    base = wid * BPW
    pltpu.sync_copy(items_h.at[pl.ds(base, BPW)], iidx)
    rowoff = lax.iota(jnp.int32, F) * CHB

    for c in range(NCH):
        r0 = c * CHB

        def bump(g, carry, r0=r0):
            it = iidx[pl.ds(r0 + g * F, F)]
            for k in range(F):
                kidxi[pl.ds(k * CHB + g * F, F)] = it + (k * NI)
            return carry

        lax.fori_loop(0, CCK, bump, 0)
        c0 = pltpu.async_copy(ig_h.at[kidxi], ig_p, sem)
        c1 = pltpu.async_copy(im_h.at[kidxi], im_p, sem)
        c0.wait()
        c1.wait()

        def asm(i, carry):
            idxv = rowoff + i
            ig_s[i] = plsc.load_gather(ig_p, [idxv])
            im_s[i] = plsc.load_gather(im_p, [idxv])
            return carry

        lax.fori_loop(0, CHB, asm, 0)
        pltpu.sync_copy(ig_s, igr_o.at[pl.ds(base + r0, CHB)])
        pltpu.sync_copy(im_s, im_o.at[pl.ds(base + r0, CHB)])


_sc_items = functools.partial(
    pl.kernel,
    mesh=plsc.VectorSubcoreMesh(core_axis_name="c", subcore_axis_name="s"),
    compiler_params=pltpu.CompilerParams(
        needs_layout_passes=False, use_tc_tiling_on_sc=False),
    out_type=[
        jax.ShapeDtypeStruct((B, F), jnp.float32),
        jax.ShapeDtypeStruct((B, F), jnp.float32),
    ],
    scratch_types=[
        pltpu.VMEM((BPW,), jnp.int32),
        pltpu.VMEM((F * CHB,), jnp.int32),
        pltpu.VMEM((F * CHB,), jnp.float32),
        pltpu.VMEM((F * CHB,), jnp.float32),
        pltpu.VMEM((CHB, F), jnp.float32),
        pltpu.VMEM((CHB, F), jnp.float32),
        pltpu.SemaphoreType.DMA,
    ],
)(_sc_items_body)


def _user_gather_loop(users_h, planes, uidx, kidxu1, bufs, sems, emit):
    def bump(g, carry):
        s = pl.ds(g * F, F)
        kidxu1[s] = uidx[s] + NUP
        return carry

    lax.fori_loop(0, BPW // F, bump, 0)

    def fire(c):
        b = c % 2
        r0 = c * CHB
        u0 = uidx.at[pl.ds(r0, CHB)]
        u1 = kidxu1.at[pl.ds(r0, CHB)]
        cs = []
        for r in range(8):
            cs.append(pltpu.async_copy(
                planes[r].at[u0], bufs[b].at[pl.ds(r * CHB, CHB)], sems[b]))
            cs.append(pltpu.async_copy(
                planes[r].at[u1],
                bufs[b].at[pl.ds((8 + r) * CHB, CHB)], sems[b]))
        return cs

    pending = fire(0)
    for c in range(NCH):
        for cp in pending:
            cp.wait()
        if c + 1 < NCH:
            pending = fire(c + 1)
        emit(c, c % 2)


def _sc_ug_body(users_h, igr_h, *rest):
    planes = rest[0:8]
    gmf_o = rest[8]
    (uidx, kidxu1, p0, p1, ig_r, gm_s, sem0, sem1) = rest[9:]
    bufs, sems = (p0, p1), (sem0, sem1)

    wid = lax.axis_index("s") * NC + lax.axis_index("c")
    base = wid * BPW
    pltpu.sync_copy(users_h.at[pl.ds(base, BPW)], uidx)
    rowoff = lax.iota(jnp.int32, F) * CHB

    def emit(c, b):
        r0 = c * CHB
        pltpu.sync_copy(igr_h.at[pl.ds(base + r0, CHB)], ig_r)

        def asm(i, carry, b=b):
            ug = plsc.load_gather(bufs[b], [rowoff + i])
            gm_s[i] = ug * ig_r[i]
            return carry

        lax.fori_loop(0, CHB, asm, 0)
        pltpu.sync_copy(gm_s, gmf_o.at[pl.ds(base + r0, CHB)])

    _user_gather_loop(users_h, planes, uidx, kidxu1, bufs, sems, emit)


def _sc_um_body(users_h, *rest):
    planes = rest[0:8]
    um_o = rest[8]
    (uidx, kidxu1, p0, p1, um_s, sem0, sem1) = rest[9:]
    bufs, sems = (p0, p1), (sem0, sem1)

    wid = lax.axis_index("s") * NC + lax.axis_index("c")
    base = wid * BPW
    pltpu.sync_copy(users_h.at[pl.ds(base, BPW)], uidx)
    rowoff = lax.iota(jnp.int32, F) * CHB

    def emit(c, b):
        r0 = c * CHB

        def asm(i, carry, b=b):
            um_s[i] = plsc.load_gather(bufs[b], [rowoff + i])
            return carry

        lax.fori_loop(0, CHB, asm, 0)
        pltpu.sync_copy(um_s, um_o.at[pl.ds(base + r0, CHB)])

    _user_gather_loop(users_h, planes, uidx, kidxu1, bufs, sems, emit)


_ug_scratch = [
    pltpu.VMEM((BPW,), jnp.int32),
    pltpu.VMEM((BPW,), jnp.int32),
    pltpu.VMEM((F * CHB,), jnp.float32),
    pltpu.VMEM((F * CHB,), jnp.float32),
    pltpu.VMEM((CHB, F), jnp.float32),
    pltpu.VMEM((CHB, F), jnp.float32),
    pltpu.SemaphoreType.DMA,
    pltpu.SemaphoreType.DMA,
]

_sc_ug = functools.partial(
    pl.kernel,
    mesh=plsc.VectorSubcoreMesh(core_axis_name="c", subcore_axis_name="s"),
    compiler_params=pltpu.CompilerParams(
        needs_layout_passes=False, use_tc_tiling_on_sc=False),
    out_type=[jax.ShapeDtypeStruct((B, F), jnp.float32)],
    scratch_types=list(_ug_scratch),
)(_sc_ug_body)

_sc_um = functools.partial(
    pl.kernel,
    mesh=plsc.VectorSubcoreMesh(core_axis_name="c", subcore_axis_name="s"),
    compiler_params=pltpu.CompilerParams(
        needs_layout_passes=False, use_tc_tiling_on_sc=False),
    out_type=[jax.ShapeDtypeStruct((B, F), jnp.float32)],
    scratch_types=[
        pltpu.VMEM((BPW,), jnp.int32),
        pltpu.VMEM((BPW,), jnp.int32),
        pltpu.VMEM((F * CHB,), jnp.float32),
        pltpu.VMEM((F * CHB,), jnp.float32),
        pltpu.VMEM((CHB, F), jnp.float32),
        pltpu.SemaphoreType.DMA,
        pltpu.SemaphoreType.DMA,
    ],
)(_sc_um_body)


BM = 2048


def _tc_body(gmf_ref, um_ref, im_ref, w1_ref, b1_ref, w2_ref, b2_ref,
             wog_ref, woh_ref, bo_ref, out_ref):
    mlp_in = jnp.concatenate([um_ref[...], im_ref[...]], axis=1)
    h = jnp.dot(mlp_in, w1_ref[...], preferred_element_type=jnp.float32)
    h = jnp.maximum(h + b1_ref[...], 0.0)
    h = jnp.dot(h, w2_ref[...], preferred_element_type=jnp.float32)
    h = jnp.maximum(h + b2_ref[...], 0.0)
    s = jnp.dot(gmf_ref[...], wog_ref[...], preferred_element_type=jnp.float32)
    s = s + jnp.dot(h, woh_ref[...], preferred_element_type=jnp.float32)
    out_ref[...] = s + bo_ref[...]


def _tc_mlp(gmf, um, im, W1, b1, W2, b2, Wo, bo):
    grid = (B // BM,)
    full = lambda shape: pl.BlockSpec(shape, lambda i: (0, 0))
    return pl.pallas_call(
        _tc_body,
        grid=grid,
        in_specs=[
            pl.BlockSpec((BM, F), lambda i: (i, 0)),
            pl.BlockSpec((BM, F), lambda i: (i, 0)),
            pl.BlockSpec((BM, F), lambda i: (i, 0)),
            full((2 * F, 2 * F)),
            full((1, 2 * F)),
            full((2 * F, F)),
            full((1, F)),
            full((F, 1)),
            full((F, 1)),
            full((1, 1)),
        ],
        out_specs=pl.BlockSpec((BM, 1), lambda i: (i, 0)),
        out_shape=jax.ShapeDtypeStruct((B, 1), jnp.float32),
    )(gmf, um, im, W1, b1.reshape(1, -1), W2, b2.reshape(1, -1),
      Wo[:F], Wo[F:], bo.reshape(1, 1))


def kernel(users, items, user_gmf, item_gmf, user_mlp, item_mlp,
           W1, b1, W2, b2, Wo, bo):
    users = users.astype(jnp.int32)
    items = items.astype(jnp.int32)
    ig_f = item_gmf.T.reshape(-1)
    im_f = item_mlp.T.reshape(-1)
    igr, im = _sc_items(items, ig_f, im_f)
    ug_planes = _repack1(user_gmf.T)
    gmf = _sc_ug(users, igr, *ug_planes)
    um_planes = _repack1(user_mlp.T)
    um = _sc_um(users, *um_planes)
    if isinstance(gmf, (list, tuple)):
        gmf = gmf[0]
    if isinstance(um, (list, tuple)):
        um = um[0]
    scores = _tc_mlp(gmf, um, im, W1, b1, W2, b2, Wo, bo)
    return scores[:, 0]


# speedup vs baseline: 1.0243x; 1.0234x over previous
import functools

import jax
import jax.numpy as jnp
from jax import lax
from jax.experimental import pallas as pl
from jax.experimental.pallas import tpu as pltpu
from jax.experimental.pallas import tpu_sc as plsc

F = 16
B = 16384
NU = 1000000
NI = 100000
NC = 2
NS = 16
NW = NC * NS
BPW = B // NW
CHB = 256
NCH = BPW // CHB
CCK = CHB // F

TCC = 131072
NUB = (NU + TCC - 1) // TCC
NUP = NUB * TCC


def _repack_body(a_ref, *out_refs):
    for r in range(8):
        out_refs[r][...] = a_ref[r, :]


def _repack1(a_t):
    blk = lambda: pl.BlockSpec((TCC,), lambda kk, j: (kk * NUB + j,))
    return pl.pallas_call(
        _repack_body,
        grid=(2, NUB),
        in_specs=[pl.BlockSpec((8, TCC), lambda kk, j: (kk, j))],
        out_specs=[blk() for _ in range(8)],
        out_shape=[jax.ShapeDtypeStruct((2 * NUP,), jnp.float32)
                   for _ in range(8)],
    )(a_t)


def _sc_items_body(items_h, ig_h, im_h,
                   igr_o, im_o,
                   iidx, kidxi, ig_p, im_p, ig_s, im_s, sem):
    wid = lax.axis_index("s") * NC + lax.axis_index("c")
    base = wid * BPW
    pltpu.sync_copy(items_h.at[pl.ds(base, BPW)], iidx)
    rowoff = lax.iota(jnp.int32, F) * CHB

    for c in range(NCH):
        r0 = c * CHB

        def bump(g, carry, r0=r0):
            it = iidx[pl.ds(r0 + g * F, F)]
            for k in range(F):
                kidxi[pl.ds(k * CHB + g * F, F)] = it + (k * NI)
            return carry

        lax.fori_loop(0, CCK, bump, 0)
        c0 = pltpu.async_copy(ig_h.at[kidxi], ig_p, sem)
        c1 = pltpu.async_copy(im_h.at[kidxi], im_p, sem)
        c0.wait()
        c1.wait()

        def asm(i, carry):
            idxv = rowoff + i
            ig_s[i] = plsc.load_gather(ig_p, [idxv])
            im_s[i] = plsc.load_gather(im_p, [idxv])
            return carry

        lax.fori_loop(0, CHB, asm, 0)
        pltpu.sync_copy(ig_s, igr_o.at[pl.ds(base + r0, CHB)])
        pltpu.sync_copy(im_s, im_o.at[pl.ds(base + r0, CHB)])


_sc_items = functools.partial(
    pl.kernel,
    mesh=plsc.VectorSubcoreMesh(core_axis_name="c", subcore_axis_name="s"),
    compiler_params=pltpu.CompilerParams(
        needs_layout_passes=False, use_tc_tiling_on_sc=False),
    out_type=[
        jax.ShapeDtypeStruct((B, F), jnp.float32),
        jax.ShapeDtypeStruct((B, F), jnp.float32),
    ],
    scratch_types=[
        pltpu.VMEM((BPW,), jnp.int32),
        pltpu.VMEM((F * CHB,), jnp.int32),
        pltpu.VMEM((F * CHB,), jnp.float32),
        pltpu.VMEM((F * CHB,), jnp.float32),
        pltpu.VMEM((CHB, F), jnp.float32),
        pltpu.VMEM((CHB, F), jnp.float32),
        pltpu.SemaphoreType.DMA,
    ],
)(_sc_items_body)


def _user_gather_loop(users_h, planes, uidx, kidxu1, bufs, sems, emit):
    def bump(g, carry):
        s = pl.ds(g * F, F)
        kidxu1[s] = uidx[s] + NUP
        return carry

    lax.fori_loop(0, BPW // F, bump, 0)

    def fire(c):
        b = c % 2
        r0 = c * CHB
        u0 = uidx.at[pl.ds(r0, CHB)]
        u1 = kidxu1.at[pl.ds(r0, CHB)]
        cs = []
        for r in range(8):
            cs.append(pltpu.async_copy(
                planes[r].at[u0], bufs[b].at[pl.ds(r * CHB, CHB)], sems[b]))
            cs.append(pltpu.async_copy(
                planes[r].at[u1],
                bufs[b].at[pl.ds((8 + r) * CHB, CHB)], sems[b]))
        return cs

    pending = fire(0)
    for c in range(NCH):
        for cp in pending:
            cp.wait()
        if c + 1 < NCH:
            pending = fire(c + 1)
        emit(c, c % 2)


def _sc_ug_body(users_h, igr_h, *rest):
    planes = rest[0:8]
    gmf_o = rest[8]
    (uidx, kidxu1, p0, p1, ig_r, gm_s, sem0, sem1) = rest[9:]
    bufs, sems = (p0, p1), (sem0, sem1)

    wid = lax.axis_index("s") * NC + lax.axis_index("c")
    base = wid * BPW
    pltpu.sync_copy(users_h.at[pl.ds(base, BPW)], uidx)
    rowoff = lax.iota(jnp.int32, F) * CHB

    def emit(c, b):
        r0 = c * CHB
        pltpu.sync_copy(igr_h.at[pl.ds(base + r0, CHB)], ig_r)

        def asm(i, carry, b=b):
            ug = plsc.load_gather(bufs[b], [rowoff + i])
            gm_s[i] = ug * ig_r[i]
            return carry

        lax.fori_loop(0, CHB, asm, 0)
        pltpu.sync_copy(gm_s, gmf_o.at[pl.ds(base + r0, CHB)])

    _user_gather_loop(users_h, planes, uidx, kidxu1, bufs, sems, emit)


def _sc_um_body(users_h, *rest):
    planes = rest[0:8]
    um_o = rest[8]
    (uidx, kidxu1, p0, p1, um_s, sem0, sem1) = rest[9:]
    bufs, sems = (p0, p1), (sem0, sem1)

    wid = lax.axis_index("s") * NC + lax.axis_index("c")
    base = wid * BPW
    pltpu.sync_copy(users_h.at[pl.ds(base, BPW)], uidx)
    rowoff = lax.iota(jnp.int32, F) * CHB

    def emit(c, b):
        r0 = c * CHB

        def asm(i, carry, b=b):
            um_s[i] = plsc.load_gather(bufs[b], [rowoff + i])
            return carry

        lax.fori_loop(0, CHB, asm, 0)
        pltpu.sync_copy(um_s, um_o.at[pl.ds(base + r0, CHB)])

    _user_gather_loop(users_h, planes, uidx, kidxu1, bufs, sems, emit)


_ug_scratch = [
    pltpu.VMEM((BPW,), jnp.int32),
    pltpu.VMEM((BPW,), jnp.int32),
    pltpu.VMEM((F * CHB,), jnp.float32),
    pltpu.VMEM((F * CHB,), jnp.float32),
    pltpu.VMEM((CHB, F), jnp.float32),
    pltpu.VMEM((CHB, F), jnp.float32),
    pltpu.SemaphoreType.DMA,
    pltpu.SemaphoreType.DMA,
]

_sc_ug = functools.partial(
    pl.kernel,
    mesh=plsc.VectorSubcoreMesh(core_axis_name="c", subcore_axis_name="s"),
    compiler_params=pltpu.CompilerParams(
        needs_layout_passes=False, use_tc_tiling_on_sc=False),
    out_type=[jax.ShapeDtypeStruct((B, F), jnp.float32)],
    scratch_types=list(_ug_scratch),
)(_sc_ug_body)

_sc_um = functools.partial(
    pl.kernel,
    mesh=plsc.VectorSubcoreMesh(core_axis_name="c", subcore_axis_name="s"),
    compiler_params=pltpu.CompilerParams(
        needs_layout_passes=False, use_tc_tiling_on_sc=False),
    out_type=[jax.ShapeDtypeStruct((B, F), jnp.float32)],
    scratch_types=[
        pltpu.VMEM((BPW,), jnp.int32),
        pltpu.VMEM((BPW,), jnp.int32),
        pltpu.VMEM((F * CHB,), jnp.float32),
        pltpu.VMEM((F * CHB,), jnp.float32),
        pltpu.VMEM((CHB, F), jnp.float32),
        pltpu.SemaphoreType.DMA,
        pltpu.SemaphoreType.DMA,
    ],
)(_sc_um_body)


BM = 2048


def _tc_body(gmf_ref, um_ref, im_ref, w1_ref, b1_ref, w2_ref, b2_ref,
             wog_ref, woh_ref, bo_ref, out_ref):
    mlp_in = jnp.concatenate([um_ref[...], im_ref[...]], axis=1)
    h = jnp.dot(mlp_in, w1_ref[...], preferred_element_type=jnp.float32)
    h = jnp.maximum(h + b1_ref[...], 0.0)
    h = jnp.dot(h, w2_ref[...], preferred_element_type=jnp.float32)
    h = jnp.maximum(h + b2_ref[...], 0.0)
    s = jnp.dot(gmf_ref[...], wog_ref[...], preferred_element_type=jnp.float32)
    s = s + jnp.dot(h, woh_ref[...], preferred_element_type=jnp.float32)
    out_ref[...] = s + bo_ref[...]


def _tc_mlp(gmf, um, im, W1, b1, W2, b2, Wo, bo):
    grid = (B // BM,)
    full = lambda shape: pl.BlockSpec(shape, lambda i: (0, 0))
    return pl.pallas_call(
        _tc_body,
        grid=grid,
        in_specs=[
            pl.BlockSpec((BM, F), lambda i: (i, 0)),
            pl.BlockSpec((BM, F), lambda i: (i, 0)),
            pl.BlockSpec((BM, F), lambda i: (i, 0)),
            full((2 * F, 2 * F)),
            full((1, 2 * F)),
            full((2 * F, F)),
            full((1, F)),
            full((F, 1)),
            full((F, 1)),
            full((1, 1)),
        ],
        out_specs=pl.BlockSpec((BM, 1), lambda i: (i, 0)),
        out_shape=jax.ShapeDtypeStruct((B, 1), jnp.float32),
    )(gmf, um, im, W1, b1.reshape(1, -1), W2, b2.reshape(1, -1),
      Wo[:F], Wo[F:], bo.reshape(1, 1))


def kernel(users, items, user_gmf, item_gmf, user_mlp, item_mlp,
           W1, b1, W2, b2, Wo, bo):
    users = users.astype(jnp.int32)
    items = items.astype(jnp.int32)
    ig_f = item_gmf.T.reshape(-1)
    im_f = item_mlp.T.reshape(-1)
    igr, im = _sc_items(items, ig_f, im_f)
    ug_planes = _repack1(user_gmf.T)
    gmf = _sc_ug(users, igr, *ug_planes)
    um_planes = _repack1(user_mlp.T)
    um = _sc_um(users, *um_planes)
    if isinstance(gmf, (list, tuple)):
        gmf = gmf[0]
    if isinstance(um, (list, tuple)):
        um = um[0]
    scores = _tc_mlp(gmf, um, im, W1, b1, W2, b2, Wo, bo)
    return scores[:, 0]
